# baseline (speedup 1.0000x reference)
import functools
import math

import jax
import jax.numpy as jnp
from jax import lax
from jax.experimental import pallas as pl
from jax.experimental.pallas import tpu as pltpu
from jax.experimental.pallas import tpu_sc as plsc

INPUT_DIM = 64
NUM_PROJS = 7
NUM_BUCKETS = 1 << NUM_PROJS
BLOCK_SIZE = 256
SAMPLE_SIZE = 256
N_SEQ = 8192
NUM_BLOCKS = N_SEQ // BLOCK_SIZE
RANK_CHUNK = 128


def _rank_of_builder(pd):
    lane = lax.broadcasted_iota(jnp.int32, (N_SEQ, NUM_BUCKETS), 1)
    jr = lax.broadcasted_iota(jnp.int32, (NUM_BUCKETS, NUM_BUCKETS), 0)
    wrep = jnp.where(jr < NUM_PROJS,
                     1 << jnp.minimum(jr, NUM_PROJS - 1),
                     0).astype(jnp.bfloat16)
    r = lax.broadcasted_iota(jnp.int32, (RANK_CHUNK, RANK_CHUNK), 0)
    c = lax.broadcasted_iota(jnp.int32, (RANK_CHUNK, RANK_CHUNK), 1)
    U_incl = (r <= c).astype(jnp.float32)
    br = lax.broadcasted_iota(jnp.int32, (NUM_BUCKETS, NUM_BUCKETS), 0)
    bc = lax.broadcasted_iota(jnp.int32, (NUM_BUCKETS, NUM_BUCKETS), 1)
    SU = (br < bc).astype(jnp.float32)
    ones_n = jnp.ones((1, N_SEQ), jnp.bfloat16)
    ones_c = jnp.ones((RANK_CHUNK, 1), jnp.float32)
    ones_1c = jnp.ones((1, RANK_CHUNK), jnp.float32)

    def rank_of(x):
        proj = jax.lax.dot_general(x, pd, (((1,), (0,)), ((), ())),
                                   preferred_element_type=jnp.float32)
        sgnb = (proj > 0).astype(jnp.bfloat16)
        binv_f = jax.lax.dot_general(sgnb, wrep, (((1,), (0,)), ((), ())),
                                     preferred_element_type=jnp.float32)
        binv = binv_f.astype(jnp.int32)
        h = binv ^ (binv >> 1)
        ohb = (h == lane).astype(jnp.bfloat16)
        hist = jax.lax.dot_general(ones_n, ohb, (((1,), (0,)), ((), ())),
                                   preferred_element_type=jnp.float32)
        bs = jax.lax.dot_general(hist, SU, (((1,), (0,)), ((), ())),
                                 preferred_element_type=jnp.float32)

        def chunk(i, carry):
            ohcb = ohb[i * RANK_CHUNK:(i + 1) * RANK_CHUNK, :]
            ohc = ohcb.astype(jnp.float32)
            g = jax.lax.dot_general(ohcb, ohcb, (((1,), (1,)), ((), ())),
                                    preferred_element_type=jnp.float32)
            t_row = jax.lax.dot_general(ones_1c, g * U_incl,
                                        (((1,), (0,)), ((), ())),
                                        preferred_element_type=jnp.float32)
            base_row = jax.lax.dot_general(bs + carry, ohc,
                                           (((1,), (1,)), ((), ())),
                                           preferred_element_type=jnp.float32)
            posc = t_row + base_row - 1.0
            carry = carry + jax.lax.dot_general(
                ones_1c, ohc, (((1,), (0,)), ((), ())),
                preferred_element_type=jnp.float32)
            return posc.astype(jnp.int32), carry

        carry = jnp.zeros((1, NUM_BUCKETS), jnp.float32)
        pieces = []
        for i in range(N_SEQ // RANK_CHUNK):
            posc, carry = chunk(i, carry)
            pieces.append(posc)
        return jnp.concatenate(pieces, axis=1)[0]

    return rank_of


def _hashq_body(q_ref, pd_ref, posq_ref, qpad_ref):
    rank_of = _rank_of_builder(pd_ref[...])
    posq_ref[0, 0] = rank_of(q_ref[0]) + pl.program_id(0) * N_SEQ
    zpad = jnp.zeros((N_SEQ, INPUT_DIM), jnp.float32)
    qpad_ref[0] = jnp.concatenate([q_ref[0], zpad], axis=1)


def _hashkv_body(k_ref, v_ref, pd_ref, posk_ref, kv_ref):
    rank_of = _rank_of_builder(pd_ref[...])
    posk_ref[0, 0] = rank_of(k_ref[0]) + pl.program_id(0) * N_SEQ
    kv_ref[0] = jnp.concatenate([k_ref[0], v_ref[0]], axis=1)


_QSPEC = pl.BlockSpec((1, N_SEQ, INPUT_DIM), lambda i: (i, 0, 0))
_PSPEC = pl.BlockSpec((INPUT_DIM, NUM_BUCKETS), lambda i: (0, 0))
_OSPEC = pl.BlockSpec((1, 1, N_SEQ), lambda i: (i, 0, 0))
_WSPEC = pl.BlockSpec((1, N_SEQ, 2 * INPUT_DIM), lambda i: (i, 0, 0))


def _hashq(q2, proj_pad):
    BH = q2.shape[0]
    pos_q, qpad = pl.pallas_call(
        _hashq_body,
        grid=(BH,),
        in_specs=[_QSPEC, _PSPEC],
        out_specs=[_OSPEC, _WSPEC],
        out_shape=[jax.ShapeDtypeStruct((BH, 1, N_SEQ), jnp.int32),
                   jax.ShapeDtypeStruct((BH, N_SEQ, 2 * INPUT_DIM),
                                        jnp.float32)],
    )(q2, proj_pad)
    return pos_q.reshape(BH, N_SEQ), qpad


def _hashkv(k2, v2, proj_pad):
    BH = k2.shape[0]
    pos_k, kv = pl.pallas_call(
        _hashkv_body,
        grid=(BH,),
        in_specs=[_QSPEC, _QSPEC, _PSPEC],
        out_specs=[_OSPEC, _WSPEC],
        out_shape=[jax.ShapeDtypeStruct((BH, 1, N_SEQ), jnp.int32),
                   jax.ShapeDtypeStruct((BH, N_SEQ, 2 * INPUT_DIM),
                                        jnp.float32)],
    )(k2, v2, proj_pad)
    return pos_k.reshape(BH, N_SEQ), kv


BLOCKS_PER_STEP = 8


def _attn_body(q_ref, kv_ref, sub_ref, samp_ref, out_ref):
    scale = INPUT_DIM ** (-0.5)
    sub = sub_ref[0]
    ks = sub[:, :INPUT_DIM]
    vs = sub[:, INPUT_DIM:]
    samp = samp_ref[0, 0]
    blk_of_samp = samp // BLOCK_SIZE
    neg = jnp.float32(jnp.finfo(jnp.float32).min)

    for t in range(BLOCKS_PER_STEP):
        nb = pl.program_id(1) * BLOCKS_PER_STEP + t
        qb = q_ref[0, t][:, :INPUT_DIM]
        kvb = kv_ref[0, t]
        kb = kvb[:, :INPUT_DIM]
        vb = kvb[:, INPUT_DIM:]

        s1 = jax.lax.dot_general(qb, kb, (((1,), (1,)), ((), ())),
                                 preferred_element_type=jnp.float32) * scale
        m1 = jnp.max(s1, axis=1, keepdims=True)
        p1 = jnp.exp(s1 - m1)
        l1 = jnp.sum(p1, axis=1, keepdims=True)
        a1 = jax.lax.dot_general(p1, vb, (((1,), (0,)), ((), ())),
                                 preferred_element_type=jnp.float32)
        lse1 = m1 + jnp.log(l1)

        s2 = jax.lax.dot_general(qb, ks, (((1,), (1,)), ((), ())),
                                 preferred_element_type=jnp.float32) * scale
        bias = jnp.where(blk_of_samp == nb, neg, jnp.float32(0.0))[None, :]
        s2 = s2 + bias
        m2 = jnp.max(s2, axis=1, keepdims=True)
        p2 = jnp.exp(s2 - m2)
        l2 = jnp.sum(p2, axis=1, keepdims=True)
        a2 = jax.lax.dot_general(p2, vs, (((1,), (0,)), ((), ())),
                                 preferred_element_type=jnp.float32)
        lse2 = m2 + jnp.log(l2) + jnp.float32(math.log(N_SEQ / SAMPLE_SIZE))

        c = jax.nn.sigmoid(lse1 - lse2)
        out = c * (a1 / l1) + (1.0 - c) * (a2 / l2)
        out_ref[0, t] = out


SUPER = 256
NSUP = N_SEQ // SUPER
IDXW = 128


def _sc_scatter_rows(wid, idx_v, buf_v, sem, src_hbm, dst_hbm):
    base = wid * N_SEQ
    per = SUPER // IDXW

    def step(s, _):
        pltpu.sync_copy(src_hbm.at[pl.ds(base + s * SUPER, SUPER)], buf_v)
        for p in range(per):
            pltpu.async_copy(buf_v.at[pl.ds(p * IDXW, IDXW)],
                             dst_hbm.at[idx_v.at[s * per + p]], sem).wait()
        return 0

    lax.fori_loop(0, NSUP, step, 0)


def _make_permute_q(BH):
    info = plsc.get_sparse_core_info()
    NC = info.num_cores
    mesh = plsc.VectorSubcoreMesh(core_axis_name="c", subcore_axis_name="s")
    W = 2 * INPUT_DIM

    @functools.partial(
        pl.kernel,
        out_type=[jax.ShapeDtypeStruct((BH * N_SEQ, W), jnp.float32)],
        mesh=mesh,
        scratch_types=[pltpu.VMEM((N_SEQ // IDXW, IDXW), jnp.int32),
                       pltpu.VMEM((SUPER, W), jnp.float32),
                       pltpu.SemaphoreType.DMA],
        compiler_params=pltpu.CompilerParams(needs_layout_passes=False),
    )
    def permute_q(qpad_hbm, posq_hbm, qs_hbm, idx_v, buf_v, sem):
        wid = lax.axis_index("s") * NC + lax.axis_index("c")
        pltpu.sync_copy(posq_hbm.at[wid], idx_v)
        _sc_scatter_rows(wid, idx_v, buf_v, sem, qpad_hbm, qs_hbm)

    return permute_q


def _make_permute_kv(BH):
    info = plsc.get_sparse_core_info()
    NC = info.num_cores
    mesh = plsc.VectorSubcoreMesh(core_axis_name="c", subcore_axis_name="s")
    S = SAMPLE_SIZE
    W = 2 * INPUT_DIM

    @functools.partial(
        pl.kernel,
        out_type=[jax.ShapeDtypeStruct((BH * N_SEQ, W), jnp.float32),
                  jax.ShapeDtypeStruct((BH * S, W), jnp.float32)],
        mesh=mesh,
        scratch_types=[pltpu.VMEM((N_SEQ // IDXW, IDXW), jnp.int32),
                       pltpu.VMEM((SUPER, W), jnp.float32),
                       pltpu.VMEM((S // IDXW, IDXW), jnp.int32),
                       pltpu.VMEM((S, W), jnp.float32),
                       pltpu.SemaphoreType.DMA],
        compiler_params=pltpu.CompilerParams(needs_layout_passes=False),
    )
    def permute_kv(kv_hbm, posk_hbm, samp_hbm, kvs_hbm, sub_hbm,
                   idx_v, buf_v, sidx_v, sub_v, sem):
        wid = lax.axis_index("s") * NC + lax.axis_index("c")
        pltpu.sync_copy(posk_hbm.at[wid], idx_v)
        _sc_scatter_rows(wid, idx_v, buf_v, sem, kv_hbm, kvs_hbm)

        pltpu.sync_copy(samp_hbm.at[wid], sidx_v)
        for p in range(S // IDXW):
            pltpu.async_copy(kvs_hbm.at[sidx_v.at[p]],
                             sub_v.at[pl.ds(p * IDXW, IDXW)], sem).wait()
        pltpu.sync_copy(sub_v, sub_hbm.at[pl.ds(wid * S, S)])

    return permute_kv


def _fused_attention(qs_pad, kv_sorted, kv_sub, samp):
    BH = qs_pad.shape[0]
    D = INPUT_DIM
    nb = NUM_BLOCKS
    qs4 = qs_pad.reshape(BH, nb, BLOCK_SIZE, 2 * D)
    kvs4 = kv_sorted.reshape(BH, nb, BLOCK_SIZE, 2 * D)
    grid = (BH, nb // BLOCKS_PER_STEP)
    oblk = pl.BlockSpec((1, BLOCKS_PER_STEP, BLOCK_SIZE, D),
                        lambda i, j: (i, j, 0, 0))
    kvblk = pl.BlockSpec((1, BLOCKS_PER_STEP, BLOCK_SIZE, 2 * D),
                         lambda i, j: (i, j, 0, 0))
    sub = pl.BlockSpec((1, SAMPLE_SIZE, 2 * D), lambda i, j: (i, 0, 0))
    sspec = pl.BlockSpec((1, 1, SAMPLE_SIZE), lambda i, j: (i, 0, 0))
    out = pl.pallas_call(
        _attn_body,
        grid=grid,
        in_specs=[kvblk, kvblk, sub, sspec],
        out_specs=oblk,
        out_shape=jax.ShapeDtypeStruct((BH, nb, BLOCK_SIZE, D), jnp.float32),
    )(qs4, kvs4, kv_sub, samp)
    return out.reshape(BH, N_SEQ, D)


def kernel(query, key, value, proj_dir, sampled_set):
    B, H, N, D = query.shape
    BH = B * H
    q2 = query.reshape(BH, N, D)
    k2 = key.reshape(BH, N, D)
    v2 = value.reshape(BH, N, D)
    samp2 = sampled_set.reshape(BH, SAMPLE_SIZE)

    proj_pad = jnp.zeros((INPUT_DIM, NUM_BUCKETS), jnp.float32)
    proj_pad = proj_pad.at[:, :NUM_PROJS].set(proj_dir[:INPUT_DIM])

    offs = jnp.arange(BH, dtype=jnp.int32)[:, None] * N
    samp_g = (samp2 + offs).reshape(BH, SAMPLE_SIZE // IDXW, IDXW)

    pos_q, qpad = _hashq(q2, proj_pad)
    posq3 = pos_q.reshape(BH, N // IDXW, IDXW)
    qs = _make_permute_q(BH)(qpad.reshape(BH * N, 2 * D), posq3)[0]

    pos_k, kv = _hashkv(k2, v2, proj_pad)
    posk3 = pos_k.reshape(BH, N // IDXW, IDXW)
    kvs, sub = _make_permute_kv(BH)(
        kv.reshape(BH * N, 2 * D), posk3, samp_g)

    merged = _fused_attention(qs.reshape(BH, N, 2 * D),
                              kvs.reshape(BH, N, 2 * D),
                              sub.reshape(BH, SAMPLE_SIZE, 2 * D),
                              samp2.reshape(BH, 1, SAMPLE_SIZE))

    out = jnp.take(merged.reshape(BH * N, D), pos_q.reshape(BH * N), axis=0)
    return out.reshape(B, H, N, D)


# speedup vs baseline: 1.2696x; 1.2696x over previous
import functools
import math

import jax
import jax.numpy as jnp
from jax import lax
from jax.experimental import pallas as pl
from jax.experimental.pallas import tpu as pltpu
from jax.experimental.pallas import tpu_sc as plsc

INPUT_DIM = 64
NUM_PROJS = 7
NUM_BUCKETS = 1 << NUM_PROJS
BLOCK_SIZE = 256
SAMPLE_SIZE = 256
N_SEQ = 8192
NUM_BLOCKS = N_SEQ // BLOCK_SIZE
RANK_CHUNK = 512


def _rank_of_builder(pd):
    lane = lax.broadcasted_iota(jnp.int32, (N_SEQ, NUM_BUCKETS), 1)
    jr = lax.broadcasted_iota(jnp.int32, (NUM_BUCKETS, NUM_BUCKETS), 0)
    wrep = jnp.where(jr < NUM_PROJS,
                     1 << jnp.minimum(jr, NUM_PROJS - 1),
                     0).astype(jnp.bfloat16)
    r = lax.broadcasted_iota(jnp.int32, (RANK_CHUNK, RANK_CHUNK), 0)
    c = lax.broadcasted_iota(jnp.int32, (RANK_CHUNK, RANK_CHUNK), 1)
    U_incl = (r <= c).astype(jnp.float32)
    br = lax.broadcasted_iota(jnp.int32, (NUM_BUCKETS, NUM_BUCKETS), 0)
    bc = lax.broadcasted_iota(jnp.int32, (NUM_BUCKETS, NUM_BUCKETS), 1)
    SU = (br < bc).astype(jnp.float32)
    ones_n = jnp.ones((1, N_SEQ), jnp.bfloat16)
    ones_c = jnp.ones((RANK_CHUNK, 1), jnp.float32)
    ones_1c = jnp.ones((1, RANK_CHUNK), jnp.float32)

    def rank_of(x):
        proj = jax.lax.dot_general(x, pd, (((1,), (0,)), ((), ())),
                                   preferred_element_type=jnp.float32)
        sgnb = (proj > 0).astype(jnp.bfloat16)
        binv_f = jax.lax.dot_general(sgnb, wrep, (((1,), (0,)), ((), ())),
                                     preferred_element_type=jnp.float32)
        binv = binv_f.astype(jnp.int32)
        h = binv ^ (binv >> 1)
        ohb = (h == lane).astype(jnp.bfloat16)
        hist = jax.lax.dot_general(ones_n, ohb, (((1,), (0,)), ((), ())),
                                   preferred_element_type=jnp.float32)
        bs = jax.lax.dot_general(hist, SU, (((1,), (0,)), ((), ())),
                                 preferred_element_type=jnp.float32)

        def chunk(i, carry):
            ohcb = ohb[i * RANK_CHUNK:(i + 1) * RANK_CHUNK, :]
            ohc = ohcb.astype(jnp.float32)
            g = jax.lax.dot_general(ohcb, ohcb, (((1,), (1,)), ((), ())),
                                    preferred_element_type=jnp.float32)
            t_row = jax.lax.dot_general(ones_1c, g * U_incl,
                                        (((1,), (0,)), ((), ())),
                                        preferred_element_type=jnp.float32)
            base_row = jax.lax.dot_general(bs + carry, ohc,
                                           (((1,), (1,)), ((), ())),
                                           preferred_element_type=jnp.float32)
            posc = t_row + base_row - 1.0
            carry = carry + jax.lax.dot_general(
                ones_1c, ohc, (((1,), (0,)), ((), ())),
                preferred_element_type=jnp.float32)
            return posc.astype(jnp.int32), carry

        carry = jnp.zeros((1, NUM_BUCKETS), jnp.float32)
        pieces = []
        for i in range(N_SEQ // RANK_CHUNK):
            posc, carry = chunk(i, carry)
            pieces.append(posc)
        return jnp.concatenate(pieces, axis=1)[0]

    return rank_of


def _hashq_body(q_ref, pd_ref, posq_ref, qpad_ref):
    rank_of = _rank_of_builder(pd_ref[...])
    posq_ref[0, 0] = rank_of(q_ref[0]) + pl.program_id(0) * N_SEQ
    zpad = jnp.zeros((N_SEQ, INPUT_DIM), jnp.float32)
    qpad_ref[0] = jnp.concatenate([q_ref[0], zpad], axis=1)


def _hashkv_body(k_ref, v_ref, pd_ref, posk_ref, kv_ref):
    rank_of = _rank_of_builder(pd_ref[...])
    posk_ref[0, 0] = rank_of(k_ref[0]) + pl.program_id(0) * N_SEQ
    kv_ref[0] = jnp.concatenate([k_ref[0], v_ref[0]], axis=1)


_QSPEC = pl.BlockSpec((1, N_SEQ, INPUT_DIM), lambda i: (i, 0, 0))
_PSPEC = pl.BlockSpec((INPUT_DIM, NUM_BUCKETS), lambda i: (0, 0))
_OSPEC = pl.BlockSpec((1, 1, N_SEQ), lambda i: (i, 0, 0))
_WSPEC = pl.BlockSpec((1, N_SEQ, 2 * INPUT_DIM), lambda i: (i, 0, 0))


def _hashq(q2, proj_pad):
    BH = q2.shape[0]
    pos_q, qpad = pl.pallas_call(
        _hashq_body,
        grid=(BH,),
        in_specs=[_QSPEC, _PSPEC],
        out_specs=[_OSPEC, _WSPEC],
        out_shape=[jax.ShapeDtypeStruct((BH, 1, N_SEQ), jnp.int32),
                   jax.ShapeDtypeStruct((BH, N_SEQ, 2 * INPUT_DIM),
                                        jnp.float32)],
    )(q2, proj_pad)
    return pos_q.reshape(BH, N_SEQ), qpad


def _hashkv(k2, v2, proj_pad):
    BH = k2.shape[0]
    pos_k, kv = pl.pallas_call(
        _hashkv_body,
        grid=(BH,),
        in_specs=[_QSPEC, _QSPEC, _PSPEC],
        out_specs=[_OSPEC, _WSPEC],
        out_shape=[jax.ShapeDtypeStruct((BH, 1, N_SEQ), jnp.int32),
                   jax.ShapeDtypeStruct((BH, N_SEQ, 2 * INPUT_DIM),
                                        jnp.float32)],
    )(k2, v2, proj_pad)
    return pos_k.reshape(BH, N_SEQ), kv


BLOCKS_PER_STEP = 8


def _attn_body(q_ref, kv_ref, sub_ref, samp_ref, out_ref):
    scale = INPUT_DIM ** (-0.5)
    sub = sub_ref[0]
    ks = sub[:, :INPUT_DIM]
    vs = sub[:, INPUT_DIM:]
    samp = samp_ref[0, 0]
    blk_of_samp = samp // BLOCK_SIZE
    neg = jnp.float32(jnp.finfo(jnp.float32).min)

    for t in range(BLOCKS_PER_STEP):
        nb = pl.program_id(1) * BLOCKS_PER_STEP + t
        qb = q_ref[0, t][:, :INPUT_DIM]
        kvb = kv_ref[0, t]
        kb = kvb[:, :INPUT_DIM]
        vb = kvb[:, INPUT_DIM:]

        s1 = jax.lax.dot_general(qb, kb, (((1,), (1,)), ((), ())),
                                 preferred_element_type=jnp.float32) * scale
        m1 = jnp.max(s1, axis=1, keepdims=True)
        p1 = jnp.exp(s1 - m1)
        l1 = jnp.sum(p1, axis=1, keepdims=True)
        a1 = jax.lax.dot_general(p1, vb, (((1,), (0,)), ((), ())),
                                 preferred_element_type=jnp.float32)
        lse1 = m1 + jnp.log(l1)

        s2 = jax.lax.dot_general(qb, ks, (((1,), (1,)), ((), ())),
                                 preferred_element_type=jnp.float32) * scale
        bias = jnp.where(blk_of_samp == nb, neg, jnp.float32(0.0))[None, :]
        s2 = s2 + bias
        m2 = jnp.max(s2, axis=1, keepdims=True)
        p2 = jnp.exp(s2 - m2)
        l2 = jnp.sum(p2, axis=1, keepdims=True)
        a2 = jax.lax.dot_general(p2, vs, (((1,), (0,)), ((), ())),
                                 preferred_element_type=jnp.float32)
        lse2 = m2 + jnp.log(l2) + jnp.float32(math.log(N_SEQ / SAMPLE_SIZE))

        c = jax.nn.sigmoid(lse1 - lse2)
        out = c * (a1 / l1) + (1.0 - c) * (a2 / l2)
        out_ref[0, t] = out


SUPER = 256
NSUP = N_SEQ // SUPER
IDXW = 128


def _sc_scatter_rows(wid, idx_v, buf_v, sem, src_hbm, dst_hbm):
    base = wid * N_SEQ
    per = SUPER // IDXW

    def step(s, _):
        pltpu.sync_copy(src_hbm.at[pl.ds(base + s * SUPER, SUPER)], buf_v)
        for p in range(per):
            pltpu.async_copy(buf_v.at[pl.ds(p * IDXW, IDXW)],
                             dst_hbm.at[idx_v.at[s * per + p]], sem).wait()
        return 0

    lax.fori_loop(0, NSUP, step, 0)


def _make_permute_q(BH):
    info = plsc.get_sparse_core_info()
    NC = info.num_cores
    mesh = plsc.VectorSubcoreMesh(core_axis_name="c", subcore_axis_name="s")
    W = 2 * INPUT_DIM

    @functools.partial(
        pl.kernel,
        out_type=[jax.ShapeDtypeStruct((BH * N_SEQ, W), jnp.float32)],
        mesh=mesh,
        scratch_types=[pltpu.VMEM((N_SEQ // IDXW, IDXW), jnp.int32),
                       pltpu.VMEM((SUPER, W), jnp.float32),
                       pltpu.SemaphoreType.DMA],
        compiler_params=pltpu.CompilerParams(needs_layout_passes=False),
    )
    def permute_q(qpad_hbm, posq_hbm, qs_hbm, idx_v, buf_v, sem):
        wid = lax.axis_index("s") * NC + lax.axis_index("c")
        pltpu.sync_copy(posq_hbm.at[wid], idx_v)
        _sc_scatter_rows(wid, idx_v, buf_v, sem, qpad_hbm, qs_hbm)

    return permute_q


def _make_permute_kv(BH):
    info = plsc.get_sparse_core_info()
    NC = info.num_cores
    mesh = plsc.VectorSubcoreMesh(core_axis_name="c", subcore_axis_name="s")
    S = SAMPLE_SIZE
    W = 2 * INPUT_DIM

    @functools.partial(
        pl.kernel,
        out_type=[jax.ShapeDtypeStruct((BH * N_SEQ, W), jnp.float32),
                  jax.ShapeDtypeStruct((BH * S, W), jnp.float32)],
        mesh=mesh,
        scratch_types=[pltpu.VMEM((N_SEQ // IDXW, IDXW), jnp.int32),
                       pltpu.VMEM((SUPER, W), jnp.float32),
                       pltpu.VMEM((S // IDXW, IDXW), jnp.int32),
                       pltpu.VMEM((S, W), jnp.float32),
                       pltpu.SemaphoreType.DMA],
        compiler_params=pltpu.CompilerParams(needs_layout_passes=False),
    )
    def permute_kv(kv_hbm, posk_hbm, samp_hbm, kvs_hbm, sub_hbm,
                   idx_v, buf_v, sidx_v, sub_v, sem):
        wid = lax.axis_index("s") * NC + lax.axis_index("c")
        pltpu.sync_copy(posk_hbm.at[wid], idx_v)
        _sc_scatter_rows(wid, idx_v, buf_v, sem, kv_hbm, kvs_hbm)

        pltpu.sync_copy(samp_hbm.at[wid], sidx_v)
        for p in range(S // IDXW):
            pltpu.async_copy(kvs_hbm.at[sidx_v.at[p]],
                             sub_v.at[pl.ds(p * IDXW, IDXW)], sem).wait()
        pltpu.sync_copy(sub_v, sub_hbm.at[pl.ds(wid * S, S)])

    return permute_kv


def _fused_attention(qs_pad, kv_sorted, kv_sub, samp):
    BH = qs_pad.shape[0]
    D = INPUT_DIM
    nb = NUM_BLOCKS
    qs4 = qs_pad.reshape(BH, nb, BLOCK_SIZE, 2 * D)
    kvs4 = kv_sorted.reshape(BH, nb, BLOCK_SIZE, 2 * D)
    grid = (BH, nb // BLOCKS_PER_STEP)
    oblk = pl.BlockSpec((1, BLOCKS_PER_STEP, BLOCK_SIZE, D),
                        lambda i, j: (i, j, 0, 0))
    kvblk = pl.BlockSpec((1, BLOCKS_PER_STEP, BLOCK_SIZE, 2 * D),
                         lambda i, j: (i, j, 0, 0))
    sub = pl.BlockSpec((1, SAMPLE_SIZE, 2 * D), lambda i, j: (i, 0, 0))
    sspec = pl.BlockSpec((1, 1, SAMPLE_SIZE), lambda i, j: (i, 0, 0))
    out = pl.pallas_call(
        _attn_body,
        grid=grid,
        in_specs=[kvblk, kvblk, sub, sspec],
        out_specs=oblk,
        out_shape=jax.ShapeDtypeStruct((BH, nb, BLOCK_SIZE, D), jnp.float32),
    )(qs4, kvs4, kv_sub, samp)
    return out.reshape(BH, N_SEQ, D)


def kernel(query, key, value, proj_dir, sampled_set):
    B, H, N, D = query.shape
    BH = B * H
    q2 = query.reshape(BH, N, D)
    k2 = key.reshape(BH, N, D)
    v2 = value.reshape(BH, N, D)
    samp2 = sampled_set.reshape(BH, SAMPLE_SIZE)

    proj_pad = jnp.zeros((INPUT_DIM, NUM_BUCKETS), jnp.float32)
    proj_pad = proj_pad.at[:, :NUM_PROJS].set(proj_dir[:INPUT_DIM])

    offs = jnp.arange(BH, dtype=jnp.int32)[:, None] * N
    samp_g = (samp2 + offs).reshape(BH, SAMPLE_SIZE // IDXW, IDXW)

    pos_q, qpad = _hashq(q2, proj_pad)
    posq3 = pos_q.reshape(BH, N // IDXW, IDXW)
    qs = _make_permute_q(BH)(qpad.reshape(BH * N, 2 * D), posq3)[0]

    pos_k, kv = _hashkv(k2, v2, proj_pad)
    posk3 = pos_k.reshape(BH, N // IDXW, IDXW)
    kvs, sub = _make_permute_kv(BH)(
        kv.reshape(BH * N, 2 * D), posk3, samp_g)

    merged = _fused_attention(qs.reshape(BH, N, 2 * D),
                              kvs.reshape(BH, N, 2 * D),
                              sub.reshape(BH, SAMPLE_SIZE, 2 * D),
                              samp2.reshape(BH, 1, SAMPLE_SIZE))

    out = jnp.take(merged.reshape(BH * N, D), pos_q.reshape(BH * N), axis=0)
    return out.reshape(B, H, N, D)
